# baseline (speedup 1.0000x reference)
import functools

import jax
import jax.numpy as jnp
from jax import lax
from jax.experimental import pallas as pl
from jax.experimental.pallas import tpu as pltpu
from jax.experimental.pallas import tpu_sc as plsc

_NUM_CLASSES = 1000000
_HIDDEN = 64
_BATCH = 16384
_DROP = 0.1

_NC = 2
_NS = 16
_NW = _NC * _NS
_BPW = _BATCH // _NW
_CHUNK = 128
_NCHUNK = _BPW // _CHUNK
_LANES = 16

_mesh = plsc.VectorSubcoreMesh(core_axis_name="c", subcore_axis_name="s")


@functools.partial(
    pl.kernel,
    mesh=_mesh,
    compiler_params=pltpu.CompilerParams(use_tc_tiling_on_sc=False),
    out_type=jax.ShapeDtypeStruct((_BATCH, _HIDDEN), jnp.float32),
    scratch_types=[
        pltpu.VMEM((_NCHUNK, _CHUNK), jnp.int32),
        pltpu.VMEM((_NCHUNK, _CHUNK), jnp.int32),
        pltpu.VMEM((_BPW, _HIDDEN), jnp.float32),
        pltpu.SemaphoreType.DMA,
    ],
)
def _embed_gather(table_hbm, labels_hbm, cond_hbm, out_hbm,
                  lab_v, cond_v, rows_v, sem):
    wid = lax.axis_index("s") * _NC + lax.axis_index("c")
    base = wid * _BPW
    pltpu.sync_copy(labels_hbm.at[wid], lab_v)
    pltpu.sync_copy(cond_hbm.at[wid], cond_v)
    for j in range(_NCHUNK):
        for k in range(_CHUNK // _LANES):
            sl = pl.ds(k * _LANES, _LANES)
            lab = lab_v[j, sl]
            cond = cond_v[j, sl]
            lab_v[j, sl] = jnp.where(cond != 0, _NUM_CLASSES, lab)
    copies = [
        pltpu.async_copy(table_hbm.at[lab_v.at[j]],
                         rows_v.at[pl.ds(j * _CHUNK, _CHUNK)], sem)
        for j in range(_NCHUNK)
    ]
    for c in copies:
        c.wait()
    pltpu.sync_copy(rows_v, out_hbm.at[pl.ds(base, _BPW)])


def kernel(labels, embedding_table, train=False, force_drop_ids=None):
    if force_drop_ids is not None:
        cond = force_drop_ids == 1
    else:
        drop = jax.random.bernoulli(jax.random.key(33), _DROP,
                                    (labels.shape[0],))
        cond = jnp.logical_and(jnp.asarray(train, dtype=bool), drop)
    lab = labels.astype(jnp.int32).reshape(_NW, _NCHUNK, _CHUNK)
    cond32 = cond.astype(jnp.int32).reshape(_NW, _NCHUNK, _CHUNK)
    return _embed_gather(embedding_table, lab, cond32)


# speedup vs baseline: 1.0303x; 1.0303x over previous
import functools

import jax
import jax.numpy as jnp
from jax import lax
from jax.experimental import pallas as pl
from jax.experimental.pallas import tpu as pltpu
from jax.experimental.pallas import tpu_sc as plsc

_NUM_CLASSES = 1000000
_HIDDEN = 64
_BATCH = 16384
_DROP = 0.1

_NC = 2
_NS = 16
_NW = _NC * _NS
_BPW = _BATCH // _NW
_LANES = 16
_UNROLL = 16

_mesh = plsc.VectorSubcoreMesh(core_axis_name="c", subcore_axis_name="s")


@functools.partial(
    pl.kernel,
    mesh=_mesh,
    compiler_params=pltpu.CompilerParams(use_tc_tiling_on_sc=True),
    out_type=jax.ShapeDtypeStruct((_BATCH, _HIDDEN), jnp.float32),
    scratch_types=[
        pltpu.VMEM((_BPW,), jnp.int32),
        pltpu.VMEM((_BPW,), jnp.int32),
        pltpu.SemaphoreType.DMA,
    ],
)
def _embed_gather(table_hbm, labels_hbm, cond_hbm, out_hbm,
                  lab_v, cond_v, sem):
    wid = lax.axis_index("s") * _NC + lax.axis_index("c")
    base = wid * _BPW
    pltpu.sync_copy(labels_hbm.at[pl.ds(base, _BPW)], lab_v)
    pltpu.sync_copy(cond_hbm.at[pl.ds(base, _BPW)], cond_v)
    for k in range(_BPW // _LANES):
        sl = pl.ds(k * _LANES, _LANES)
        lab_v[sl] = jnp.where(cond_v[sl] != 0, _NUM_CLASSES, lab_v[sl])

    def issue(g, carry):
        vec = lab_v[pl.ds(g * _UNROLL, _UNROLL)]
        for k in range(_UNROLL):
            idx = vec[k]
            pltpu.async_copy(table_hbm.at[idx],
                             out_hbm.at[base + g * _UNROLL + k], sem)
        return carry

    lax.fori_loop(0, _BPW // _UNROLL, issue, 0)
    pltpu.make_async_copy(table_hbm.at[pl.ds(0, _BPW)],
                          out_hbm.at[pl.ds(base, _BPW)], sem).wait()


def kernel(labels, embedding_table, train=False, force_drop_ids=None):
    if force_drop_ids is not None:
        cond = force_drop_ids == 1
    else:
        drop = jax.random.bernoulli(jax.random.key(33), _DROP,
                                    (labels.shape[0],))
        cond = jnp.logical_and(jnp.asarray(train, dtype=bool), drop)
    lab = labels.astype(jnp.int32)
    cond32 = cond.astype(jnp.int32)
    return _embed_gather(embedding_table, lab, cond32)
